# baseline (speedup 1.0000x reference)
import functools
import jax
import jax.numpy as jnp
from jax import lax
from jax.experimental import pallas as pl
from jax.experimental.pallas import tpu as pltpu
from jax.experimental.pallas import tpu_sc as plsc

N, T, P, D = 8, 48, 325, 512
TOK = N * T * P
NW = 32
TPW = TOK // NW
TPAD = 3904
TILE = 80
NFULL = TPW // TILE
TAIL = TPW - NFULL * TILE
L = 16
CH = D // L

_mesh = plsc.VectorSubcoreMesh(core_axis_name="c", subcore_axis_name="s")


@functools.partial(
    pl.kernel,
    mesh=_mesh,
    compiler_params=pltpu.CompilerParams(needs_layout_passes=False),
    out_type=jax.ShapeDtypeStruct((TOK * D,), jnp.float32),
    scratch_types=[
        pltpu.VMEM((TPAD,), jnp.float32),
        pltpu.VMEM((TPAD,), jnp.float32),
        pltpu.VMEM((TPAD,), jnp.float32),
        pltpu.VMEM((TPAD,), jnp.float32),
        pltpu.VMEM((TPAD,), jnp.int32),
        pltpu.VMEM((D,), jnp.float32),
        pltpu.VMEM((D,), jnp.float32),
        pltpu.VMEM((3 * D,), jnp.float32),
        pltpu.VMEM((TILE * D,), jnp.float32),
    ],
)
def _sc_embed(t_hbm, v_hbm, m_hbm, tw_hbm, vw_hbm, base_hbm, out_hbm,
              t_v, v_v, m_v, b_v, s_v, tw_v, vw_v, base_v, obuf):
    wid = lax.axis_index("s") * 2 + lax.axis_index("c")
    gbase = wid * TPW

    pltpu.sync_copy(t_hbm.at[pl.ds(wid * TPAD, TPAD)], t_v)
    pltpu.sync_copy(v_hbm.at[pl.ds(wid * TPAD, TPAD)], v_v)
    pltpu.sync_copy(m_hbm.at[pl.ds(wid * TPAD, TPAD)], m_v)
    pltpu.sync_copy(tw_hbm, tw_v)
    pltpu.sync_copy(vw_hbm, vw_v)
    pltpu.sync_copy(base_hbm, base_v)

    zf = jnp.zeros((L,), jnp.float32)
    s_emp = jnp.full((L,), D, jnp.int32)
    s_unm = jnp.full((L,), 2 * D, jnp.int32)
    s_val = jnp.zeros((L,), jnp.int32)
    iota16 = lax.iota(jnp.int32, L)

    def p1(j, carry):
        sl = pl.ds(j * L, L)
        v = v_v[sl]
        m = m_v[sl]
        inval = v != v
        notmon = m == zf
        b_v[sl] = jnp.where(inval | notmon, zf, v)
        s_v[sl] = jnp.where(notmon, s_unm, jnp.where(inval, s_emp, s_val))
        return carry

    lax.fori_loop(0, TPAD // L, p1, 0)

    def per_group(g, r0):
        tvec = t_v[pl.ds(g * L, L)]
        bvec = b_v[pl.ds(g * L, L)]
        svec = s_v[pl.ds(g * L, L)]
        obase = (g * L - r0) * D
        for j in range(L):
            a = tvec[j]
            b = bvec[j]
            idx0 = iota16 + svec[j]
            roff = obase + j * D

            def per_chunk(k, c2):
                sl = pl.ds(k * L, L)
                bk = plsc.load_gather(base_v, [idx0 + k * L])
                obuf[pl.ds(roff + k * L, L)] = a * tw_v[sl] + b * vw_v[sl] + bk
                return c2

            lax.fori_loop(0, CH, per_chunk, 0)
        return r0

    GPT = TILE // L

    def do_tile(t, carry):
        r0 = t * TILE
        lax.fori_loop(r0 // L, r0 // L + GPT, per_group, r0)
        pltpu.sync_copy(obuf,
                        out_hbm.at[pl.ds((gbase + r0) * D, TILE * D)])
        return carry

    lax.fori_loop(0, NFULL, do_tile, 0)

    r0t = NFULL * TILE
    lax.fori_loop(r0t // L, r0t // L + 4, per_group, r0t)
    pltpu.sync_copy(obuf.at[pl.ds(0, TAIL * D)],
                    out_hbm.at[pl.ds((gbase + r0t) * D, TAIL * D)])


def kernel(x, monitor_mask, time_emb_w, time_emb_b, value_emb_w, value_emb_b,
           empty_token, unmonitored_token):
    value = x[..., 0].reshape(NW, TPW)
    time = x[..., 1].reshape(NW, TPW)
    mon = monitor_mask.reshape(NW, TPW).astype(jnp.float32)
    pad = ((0, 0), (0, TPAD - TPW))
    tb = time_emb_b.reshape(D)
    base = jnp.concatenate([
        tb + value_emb_b.reshape(D),
        tb + empty_token,
        tb + unmonitored_token,
    ])
    out = _sc_embed(jnp.pad(time, pad).reshape(-1),
                    jnp.pad(value, pad).reshape(-1),
                    jnp.pad(mon, pad).reshape(-1),
                    time_emb_w.reshape(D), value_emb_w.reshape(D), base)
    return out.reshape(N, T, P, D)


# speedup vs baseline: 1.3524x; 1.3524x over previous
import functools
import jax
import jax.numpy as jnp
from jax import lax
from jax.experimental import pallas as pl
from jax.experimental.pallas import tpu as pltpu
from jax.experimental.pallas import tpu_sc as plsc

N, T, P, D = 8, 48, 325, 512
TOK = N * T * P
NW = 32
TPW = TOK // NW
TPAD = 3904
TILE = 80
NFULL = TPW // TILE
TAIL = TPW - NFULL * TILE
L = 16
CH = D // L
GPT = TILE // L
SUB = 8
GSLICE = 3 * D - (CH - 1) * L

_mesh = plsc.VectorSubcoreMesh(core_axis_name="c", subcore_axis_name="s")


@functools.partial(
    pl.kernel,
    mesh=_mesh,
    compiler_params=pltpu.CompilerParams(needs_layout_passes=False),
    out_type=jax.ShapeDtypeStruct((TOK * D,), jnp.float32),
    scratch_types=[
        pltpu.VMEM((TPAD,), jnp.float32),
        pltpu.VMEM((TPAD,), jnp.float32),
        pltpu.VMEM((TPAD,), jnp.float32),
        pltpu.VMEM((TPAD,), jnp.float32),
        pltpu.VMEM((TPAD,), jnp.int32),
        pltpu.VMEM((D,), jnp.float32),
        pltpu.VMEM((D,), jnp.float32),
        pltpu.VMEM((3 * D,), jnp.float32),
        pltpu.VMEM((TILE * D,), jnp.float32),
        pltpu.VMEM((TILE * D,), jnp.float32),
        pltpu.SemaphoreType.DMA,
        pltpu.SemaphoreType.DMA,
    ],
)
def _sc_embed(t_hbm, v_hbm, m_hbm, tw_hbm, vw_hbm, base_hbm, out_hbm,
              t_v, v_v, m_v, b_v, s_v, tw_v, vw_v, base_v, obuf0, obuf1,
              semA, semB):
    wid = lax.axis_index("s") * 2 + lax.axis_index("c")
    gbase = wid * TPW

    pltpu.sync_copy(t_hbm.at[pl.ds(wid * TPAD, TPAD)], t_v)
    pltpu.sync_copy(v_hbm.at[pl.ds(wid * TPAD, TPAD)], v_v)
    pltpu.sync_copy(m_hbm.at[pl.ds(wid * TPAD, TPAD)], m_v)
    pltpu.sync_copy(tw_hbm, tw_v)
    pltpu.sync_copy(vw_hbm, vw_v)
    pltpu.sync_copy(base_hbm, base_v)

    zf = jnp.zeros((L,), jnp.float32)
    s_emp = jnp.full((L,), D, jnp.int32)
    s_unm = jnp.full((L,), 2 * D, jnp.int32)
    s_val = jnp.zeros((L,), jnp.int32)
    iota16 = lax.iota(jnp.int32, L)

    def p1(j, carry):
        sl = pl.ds(j * L, L)
        v = v_v[sl]
        m = m_v[sl]
        inval = v != v
        notmon = m == zf
        b_v[sl] = jnp.where(inval | notmon, zf, v)
        s_v[sl] = jnp.where(notmon, s_unm, jnp.where(inval, s_emp, s_val))
        return carry

    lax.fori_loop(0, TPAD // L, p1, 0)

    def compute_tile(tile, obuf):
        def per_group(lg, carry):
            toff = tile * TILE + lg * L
            tvec = t_v[pl.ds(toff, L)]
            bvec = b_v[pl.ds(toff, L)]
            svec = s_v[pl.ds(toff, L)]
            obase = lg * L * D
            for sub in range(L // SUB):
                A = [jnp.full((L,), tvec[sub * SUB + j]) for j in range(SUB)]
                B = [jnp.full((L,), bvec[sub * SUB + j]) for j in range(SUB)]
                IX = [iota16 + svec[sub * SUB + j] for j in range(SUB)]
                rbase = obase + sub * SUB * D

                def per_chunk(k, c2):
                    ksl = pl.ds(k * L, L)
                    twk = tw_v[ksl]
                    vwk = vw_v[ksl]
                    win = base_v.at[pl.ds(k * L, GSLICE)]
                    for j in range(SUB):
                        bk = plsc.load_gather(win, [IX[j]])
                        obuf[pl.ds(rbase + j * D + k * L, L)] = (
                            A[j] * twk + B[j] * vwk + bk)
                    return c2

                lax.fori_loop(0, CH, per_chunk, 0, unroll=8)
            return carry

        lax.fori_loop(0, GPT, per_group, 0)

    def dma_out(obuf, tile, sem):
        return pltpu.make_async_copy(
            obuf, out_hbm.at[pl.ds((gbase + tile * TILE) * D, TILE * D)], sem)

    def wait_out(obuf, sem):
        pltpu.make_async_copy(
            obuf, out_hbm.at[pl.ds(gbase * D, TILE * D)], sem).wait()

    compute_tile(0, obuf0)
    dma_out(obuf0, 0, semA).start()
    compute_tile(1, obuf1)
    dma_out(obuf1, 1, semB).start()

    def pair(p, carry):
        wait_out(obuf0, semA)
        compute_tile(2 * p, obuf0)
        dma_out(obuf0, 2 * p, semA).start()
        wait_out(obuf1, semB)
        compute_tile(2 * p + 1, obuf1)
        dma_out(obuf1, 2 * p + 1, semB).start()
        return carry

    lax.fori_loop(1, NFULL // 2, pair, 0)

    wait_out(obuf0, semA)

    def tail_group(lg, carry):
        toff = NFULL * TILE + lg * L
        tvec = t_v[pl.ds(toff, L)]
        bvec = b_v[pl.ds(toff, L)]
        svec = s_v[pl.ds(toff, L)]
        obase = lg * L * D
        for sub in range(L // SUB):
            A = [jnp.full((L,), tvec[sub * SUB + j]) for j in range(SUB)]
            B = [jnp.full((L,), bvec[sub * SUB + j]) for j in range(SUB)]
            IX = [iota16 + svec[sub * SUB + j] for j in range(SUB)]
            rbase = obase + sub * SUB * D

            def per_chunk(k, c2):
                ksl = pl.ds(k * L, L)
                twk = tw_v[ksl]
                vwk = vw_v[ksl]
                win = base_v.at[pl.ds(k * L, GSLICE)]
                for j in range(SUB):
                    bk = plsc.load_gather(win, [IX[j]])
                    obuf0[pl.ds(rbase + j * D + k * L, L)] = (
                        A[j] * twk + B[j] * vwk + bk)
                return c2

            lax.fori_loop(0, CH, per_chunk, 0, unroll=8)
        return carry

    lax.fori_loop(0, 4, tail_group, 0)
    wait_out(obuf1, semB)
    pltpu.sync_copy(obuf0.at[pl.ds(0, TAIL * D)],
                    out_hbm.at[pl.ds((gbase + NFULL * TILE) * D, TAIL * D)])


def kernel(x, monitor_mask, time_emb_w, time_emb_b, value_emb_w, value_emb_b,
           empty_token, unmonitored_token):
    value = x[..., 0].reshape(NW, TPW)
    time = x[..., 1].reshape(NW, TPW)
    mon = monitor_mask.reshape(NW, TPW).astype(jnp.float32)
    pad = ((0, 0), (0, TPAD - TPW))
    tb = time_emb_b.reshape(D)
    base = jnp.concatenate([
        tb + value_emb_b.reshape(D),
        tb + empty_token,
        tb + unmonitored_token,
    ])
    out = _sc_embed(jnp.pad(time, pad).reshape(-1),
                    jnp.pad(value, pad).reshape(-1),
                    jnp.pad(mon, pad).reshape(-1),
                    time_emb_w.reshape(D), value_emb_w.reshape(D), base)
    return out.reshape(N, T, P, D)
